# Initial kernel scaffold; baseline (speedup 1.0000x reference)
import jax
import jax.numpy as jnp
from jax.experimental import pallas as pl


def kernel(input, Wh1, bh1, Wh2, bh2, Wh3, bh3, Wd1, bd1, Wd2, bd2, Wd3, bd3):
    raise NotImplementedError("write your pallas kernel here")


# speedup vs baseline: 1.0104x; 1.0104x over previous
import jax
import jax.numpy as jnp
from jax.experimental import pallas as pl
from jax.experimental.pallas import tpu as pltpu

B = 16384
D = 256
H1 = 64
H2 = 64
S = 2048

R = 64


def _body(x_ref, wh1, bh1, wh2, bh2, wh3, bh3, wd1, bd1, wd2, bd2, wd3, bd3,
          samp_h, gum_h, samp_d, gum_d,
          prob_h_ref, act_h_ref, prob_d_ref, act_d_ref):
    x = x_ref[...]
    col = jax.lax.broadcasted_iota(jnp.int32, (R, S), 1)

    def branch(w1, b1, w2, b2, w3, b3, samp, gum, prob_ref, act_ref):
        h = jnp.maximum(jnp.dot(x, w1[...], preferred_element_type=jnp.float32)
                        + b1[...], 0.0)
        h = jnp.maximum(jnp.dot(h, w2[...], preferred_element_type=jnp.float32)
                        + b2[...], 0.0)
        out = jax.nn.sigmoid(jnp.dot(h, w3[...], preferred_element_type=jnp.float32)
                             + b3[...])
        prob = jnp.clip(jnp.maximum(out, samp[...]), 0.01, 0.99)
        prob_ref[...] = prob
        s_lo = jnp.log(1.0 - prob) + gum[:, :S]
        s_hi = jnp.log(prob) + gum[:, S:]
        m_lo = jnp.max(s_lo, axis=1, keepdims=True)
        i_lo = jnp.min(jnp.where(s_lo == m_lo, col, 2 * S), axis=1, keepdims=True)
        m_hi = jnp.max(s_hi, axis=1, keepdims=True)
        i_hi = jnp.min(jnp.where(s_hi == m_hi, col, 2 * S), axis=1, keepdims=True)
        act_ref[...] = jnp.where(m_lo >= m_hi, i_lo, i_hi + S)

    branch(wh1, bh1, wh2, bh2, wh3, bh3, samp_h, gum_h, prob_h_ref, act_h_ref)
    branch(wd1, bd1, wd2, bd2, wd3, bd3, samp_d, gum_d, prob_d_ref, act_d_ref)


def kernel(input, Wh1, bh1, Wh2, bh2, Wh3, bh3, Wd1, bd1, Wd2, bd2, Wd3, bd3):
    key = jax.random.key(42)
    k1, k2, k3, k4 = jax.random.split(key, 4)
    samp_h = jax.random.normal(k1, (B, S), dtype=jnp.float32) * 0.25 + 0.5
    gum_h = jax.random.gumbel(k2, (B, 2 * S), dtype=jnp.float32)
    samp_d = jax.random.normal(k3, (B, S), dtype=jnp.float32) * 0.25 + 0.5
    gum_d = jax.random.gumbel(k4, (B, 2 * S), dtype=jnp.float32)

    row = lambda i: (i, 0)
    rep = lambda i: (0, 0)
    grid = B // R
    out = pl.pallas_call(
        _body,
        grid=(grid,),
        in_specs=[
            pl.BlockSpec((R, D), row),
            pl.BlockSpec((D, H1), rep), pl.BlockSpec((1, H1), rep),
            pl.BlockSpec((H1, H2), rep), pl.BlockSpec((1, H2), rep),
            pl.BlockSpec((H2, S), rep), pl.BlockSpec((1, S), rep),
            pl.BlockSpec((D, H1), rep), pl.BlockSpec((1, H1), rep),
            pl.BlockSpec((H1, H2), rep), pl.BlockSpec((1, H2), rep),
            pl.BlockSpec((H2, S), rep), pl.BlockSpec((1, S), rep),
            pl.BlockSpec((R, S), row), pl.BlockSpec((R, 2 * S), row),
            pl.BlockSpec((R, S), row), pl.BlockSpec((R, 2 * S), row),
        ],
        out_specs=[
            pl.BlockSpec((R, S), row), pl.BlockSpec((R, 1), row),
            pl.BlockSpec((R, S), row), pl.BlockSpec((R, 1), row),
        ],
        out_shape=[
            jax.ShapeDtypeStruct((B, S), jnp.float32),
            jax.ShapeDtypeStruct((B, 1), jnp.int32),
            jax.ShapeDtypeStruct((B, S), jnp.float32),
            jax.ShapeDtypeStruct((B, 1), jnp.int32),
        ],
    )(input,
      Wh1, bh1.reshape(1, H1), Wh2, bh2.reshape(1, H2), Wh3, bh3.reshape(1, S),
      Wd1, bd1.reshape(1, H1), Wd2, bd2.reshape(1, H2), Wd3, bd3.reshape(1, S),
      samp_h, gum_h, samp_d, gum_d)
    prob_h, act_h, prob_d, act_d = out
    return (prob_h, act_h.reshape(B), prob_d, act_d.reshape(B))
